# baseline (speedup 1.0000x reference)
import functools

import jax
import jax.numpy as jnp
from jax import lax
from jax.experimental import pallas as pl
from jax.experimental.pallas import tpu as pltpu
from jax.experimental.pallas import tpu_sc as plsc

D = 64
BATCH = 4096
HIST = 200
NC, NS = 2, 16
NW = NC * NS
BB = BATCH // NW
L = 16

_mesh = plsc.VectorSubcoreMesh(core_axis_name="c", subcore_axis_name="s")


@functools.partial(
    pl.kernel,
    mesh=_mesh,
    compiler_params=pltpu.CompilerParams(
        use_tc_tiling_on_sc=False,
        needs_layout_passes=False,
        disable_bounds_checks=True,
    ),
    out_type=jax.ShapeDtypeStruct((HIST, D // 8, NW, 8 * BB), jnp.float32),
    scratch_types=(
        [
            pltpu.VMEM((HIST, BB), jnp.int32),
            pltpu.VMEM((2, BB, D), jnp.float32),
            pltpu.VMEM((2, D // 8, 8 * BB), jnp.float32),
        ]
        + [pltpu.SemaphoreType.DMA] * 4
    ),
)
def _embed5(table_hbm, idxt_hbm, out_hbm, idx_v, rows_v, tiles_v, *sems):
  gsem = sems[:2]
  ssem = sems[2:]
  wid = lax.axis_index("s") * NC + lax.axis_index("c")

  pltpu.sync_copy(idxt_hbm.at[:, pl.ds(wid * BB, BB)], idx_v)

  iota = lax.iota(jnp.int32, L)
  grp = [iota + L * g for g in range(BB // L)]

  def start_gather(t, ph):
    pltpu.async_copy(table_hbm.at[idx_v.at[t]], rows_v.at[ph], gsem[ph])

  def wait_gather(t, ph):
    pltpu.make_async_copy(
        table_hbm.at[idx_v.at[t]], rows_v.at[ph], gsem[ph]
    ).wait()

  def start_out(t, ph):
    pltpu.async_copy(tiles_v.at[ph], out_hbm.at[t, :, wid], ssem[ph])

  def wait_out(t, ph):
    pltpu.make_async_copy(
        tiles_v.at[ph], out_hbm.at[t, :, wid], ssem[ph]
    ).wait()

  start_gather(0, 0)

  def pair_body(t2, carry):
    for ph in range(2):
      t = t2 * 2 + ph
      wait_gather(t, ph)

      @pl.when(t + 1 < HIST)
      def _():
        start_gather(t + 1, 1 - ph)

      @pl.when(t >= 2)
      def _():
        wait_out(t - 2, ph)

      tile = tiles_v.at[ph]
      rows = rows_v.at[ph]
      pipe = []
      for d in range(D):
        col = jnp.full((L,), d, jnp.int32)
        for g in range(BB // L):
          pipe.append((d, g, plsc.load_gather(rows, [grp[g], col])))
          if len(pipe) == 8:
            dq, gq, xq = pipe.pop(0)
            tile[dq // 8, pl.ds((dq % 8) * BB + L * gq, L)] = xq
      for dq, gq, xq in pipe:
        tile[dq // 8, pl.ds((dq % 8) * BB + L * gq, L)] = xq
      start_out(t, ph)
    return carry

  lax.fori_loop(0, HIST // 2, pair_body, 0)

  wait_out(HIST - 2, 0)
  wait_out(HIST - 1, 1)


def kernel(input, table):
  idxt = input.T
  o4 = _embed5(table, idxt)
  o5 = o4.reshape(HIST, D // 8, NW, 8, BB)
  return o5.transpose(2, 4, 0, 1, 3).reshape(BATCH, HIST, D)


# speedup vs baseline: 1.6226x; 1.6226x over previous
import functools

import jax
import jax.numpy as jnp
from jax import lax
from jax.experimental import pallas as pl
from jax.experimental.pallas import tpu as pltpu
from jax.experimental.pallas import tpu_sc as plsc

D = 64
BATCH = 4096
HIST = 200
NC, NS = 2, 16
NW = NC * NS
BB = BATCH // NW
L = 16
BP = BB + 1

_mesh = plsc.VectorSubcoreMesh(core_axis_name="c", subcore_axis_name="s")


@functools.partial(
    pl.kernel,
    mesh=_mesh,
    compiler_params=pltpu.CompilerParams(
        use_tc_tiling_on_sc=False,
        needs_layout_passes=False,
        disable_bounds_checks=True,
    ),
    out_type=jax.ShapeDtypeStruct((HIST, D // 8, NW, 8, BB), jnp.float32),
    scratch_types=(
        [
            pltpu.VMEM((HIST, BB), jnp.int32),
            pltpu.VMEM((2, BB, D), jnp.float32),
            pltpu.VMEM((2, D // 8, 8, BP), jnp.float32),
        ]
        + [pltpu.SemaphoreType.DMA] * 4
    ),
)
def _embed5(table_hbm, idxt_hbm, out_hbm, idx_v, rows_v, tiles_v, *sems):
  gsem = sems[:2]
  ssem = sems[2:]
  wid = lax.axis_index("s") * NC + lax.axis_index("c")

  pltpu.sync_copy(idxt_hbm.at[:, pl.ds(wid * BB, BB)], idx_v)

  iota = lax.iota(jnp.int32, L)
  rows_idx = [(L * k + iota) >> 3 for k in range(D // L)]
  cols_idx = [(L * k + iota) & 7 for k in range(D // L)]

  def start_gather(t, ph):
    pltpu.async_copy(table_hbm.at[idx_v.at[t]], rows_v.at[ph], gsem[ph])

  def wait_gather(t, ph):
    pltpu.make_async_copy(
        table_hbm.at[idx_v.at[t]], rows_v.at[ph], gsem[ph]
    ).wait()

  def start_out(t, ph):
    pltpu.async_copy(
        tiles_v.at[ph, :, :, pl.ds(0, BB)], out_hbm.at[t, :, wid], ssem[ph]
    )

  def wait_out(t, ph):
    pltpu.make_async_copy(
        tiles_v.at[ph, :, :, pl.ds(0, BB)], out_hbm.at[t, :, wid], ssem[ph]
    ).wait()

  start_gather(0, 0)

  def pair_body(t2, carry):
    for ph in range(2):
      t = t2 * 2 + ph
      wait_gather(t, ph)

      @pl.when(t + 1 < HIST)
      def _():
        start_gather(t + 1, 1 - ph)

      @pl.when(t >= 2)
      def _():
        wait_out(t - 2, ph)

      tile = tiles_v.at[ph]
      pipe = []
      for b in range(BB):
        lane = jnp.full((L,), b, jnp.int32)
        for k in range(D // L):
          pipe.append((k, lane, rows_v[ph, b, pl.ds(L * k, L)]))
          if len(pipe) == 8:
            kq, lq, xq = pipe.pop(0)
            plsc.store_scatter(tile, [rows_idx[kq], cols_idx[kq], lq], xq)
      for kq, lq, xq in pipe:
        plsc.store_scatter(tile, [rows_idx[kq], cols_idx[kq], lq], xq)
      start_out(t, ph)
    return carry

  lax.fori_loop(0, HIST // 2, pair_body, 0)

  wait_out(HIST - 2, 0)
  wait_out(HIST - 1, 1)


def kernel(input, table):
  idxt = input.T
  o5 = _embed5(table, idxt)
  return o5.transpose(2, 4, 0, 1, 3).reshape(BATCH, HIST, D)
